# baseline (speedup 1.0000x reference)
import jax
import jax.numpy as jnp
from jax import lax
from jax.experimental import pallas as pl
from jax.experimental.pallas import tpu as pltpu
from jax.experimental.pallas import tpu_sc as plsc

_B, _S, _G, _F = 4096, 50, 1, 2
_D = 32
_FIELD_OFFSET = 1000000

_M = _B * _S * _G
_CHUNK = 128
_NC, _NS = 2, 16
_NW = _NC * _NS
_RPW = _M // _CHUNK // _NW


def _gather_body(idx0_hbm, idx1_hbm, table_hbm, emb0_hbm, emb1_hbm,
                 idx0_v, idx1_v, rows0, rows1, sem0, sem1):
    wid = lax.axis_index("s") * _NC + lax.axis_index("c")
    base = wid * _RPW
    pltpu.sync_copy(idx0_hbm.at[wid], idx0_v)
    pltpu.sync_copy(idx1_hbm.at[wid], idx1_v)

    offs = jnp.full((16,), _FIELD_OFFSET, dtype=jnp.int32)

    def add_offs(j, carry):
        for k in range(_CHUNK // 16):
            sl = pl.ds(k * 16, 16)
            idx1_v[j, sl] = idx1_v[j, sl] + offs
        return carry

    lax.fori_loop(0, _RPW, add_offs, 0)

    def fetch(j, idx_v, rows, sem, emb_hbm):
        pltpu.async_copy(table_hbm.at[idx_v.at[j]], rows, sem).wait()
        pltpu.sync_copy(rows, emb_hbm.at[pl.ds((base + j) * _CHUNK, _CHUNK)])

    def chunk(j, carry):
        fetch(j, idx0_v, rows0, sem0, emb0_hbm)
        fetch(j, idx1_v, rows1, sem1, emb1_hbm)
        return carry

    lax.fori_loop(0, _RPW, chunk, 0)


_gather = pl.kernel(
    _gather_body,
    out_type=(
        jax.ShapeDtypeStruct((_M, _D), jnp.float32),
        jax.ShapeDtypeStruct((_M, _D), jnp.float32),
    ),
    mesh=plsc.VectorSubcoreMesh(core_axis_name="c", subcore_axis_name="s"),
    compiler_params=pltpu.CompilerParams(use_tc_tiling_on_sc=False),
    scratch_types=[
        pltpu.VMEM((_RPW, _CHUNK), jnp.int32),
        pltpu.VMEM((_RPW, _CHUNK), jnp.int32),
        pltpu.VMEM((_CHUNK, _D), jnp.float32),
        pltpu.VMEM((_CHUNK, _D), jnp.float32),
        pltpu.SemaphoreType.DMA,
        pltpu.SemaphoreType.DMA,
    ],
)


def _proj_body(e0_ref, e1_ref, w0_ref, w1_ref, b_ref, out_ref):
    dn = (((1,), (1,)), ((), ()))
    out_ref[...] = (
        lax.dot_general(e0_ref[...], w0_ref[...], dn,
                        preferred_element_type=jnp.float32)
        + lax.dot_general(e1_ref[...], w1_ref[...], dn,
                          preferred_element_type=jnp.float32)
        + b_ref[...]
    )


_BLK = 2048


def _proj(e0, e1, w0, w1, b2):
    return pl.pallas_call(
        _proj_body,
        grid=(_M // _BLK,),
        in_specs=[
            pl.BlockSpec((_BLK, _D), lambda i: (i, 0)),
            pl.BlockSpec((_BLK, _D), lambda i: (i, 0)),
            pl.BlockSpec((_D, _D), lambda i: (0, 0)),
            pl.BlockSpec((_D, _D), lambda i: (0, 0)),
            pl.BlockSpec((1, _D), lambda i: (0, 0)),
        ],
        out_specs=pl.BlockSpec((_BLK, _D), lambda i: (i, 0)),
        out_shape=jax.ShapeDtypeStruct((_M, _D), jnp.float32),
    )(e0, e1, w0, w1, b2)


def kernel(x, table, W, b):
    idx0 = x[:, :, :, 0].reshape(_NW, _RPW, _CHUNK)
    idx1 = x[:, :, :, 1].reshape(_NW, _RPW, _CHUNK)
    emb0, emb1 = _gather(idx0, idx1, table)
    out = _proj(emb0, emb1, W[:, :_D], W[:, _D:], b.reshape(1, _D))
    return out.reshape(_B, _S, _G, _D)
